# baseline (speedup 1.0000x reference)
import functools

import jax
import jax.numpy as jnp
from jax import lax
from jax.experimental import pallas as pl
from jax.experimental.pallas import tpu as pltpu
from jax.experimental.pallas import tpu_sc as plsc

E = 64
B = 4
D_OUT = 1024
D_IN = 1024
D = D_OUT * D_IN

L = 16
NC = 2
NS = 16
NW = NC * NS
WPB = NW // B
PART = D // WPB
R = 2048
NR = 8
ROWS_PER_E = D // R
GROUP = NR * R
G = PART // GROUP
OROWS = GROUP // D_IN
UNROLL = 8


def _shuf_max(v, sbuf, iota):
    for sh in (1, 2, 4, 8):
        sbuf[...] = v
        v = jnp.maximum(v, plsc.load_gather(sbuf, [iota ^ sh]))
    return v


def _shuf_min_i32(v, sbuf, iota):
    for sh in (1, 2, 4, 8):
        sbuf[...] = v
        v = jnp.minimum(v, plsc.load_gather(sbuf, [iota ^ sh]))
    return v


def _routing(lbuf, fsc, isc, iota):
    vs = [lbuf[pl.ds(j * L, L)] for j in range(E // L)]

    m = vs[0]
    for v in vs[1:]:
        m = jnp.maximum(m, v)
    m1v = _shuf_max(m, fsc, iota)

    cmin = jnp.full((L,), E, jnp.int32)
    for j, v in enumerate(vs):
        cmin = jnp.minimum(cmin, jnp.where(v == m1v, iota + (j * L), E))
    i1v = _shuf_min_i32(cmin, isc, iota)

    neg_inf = jnp.float32(-jnp.inf)
    vs2 = [jnp.where(iota + (j * L) == i1v, neg_inf, v) for j, v in enumerate(vs)]
    m2 = vs2[0]
    for v in vs2[1:]:
        m2 = jnp.maximum(m2, v)
    m2v = _shuf_max(m2, fsc, iota)

    cmin2 = jnp.full((L,), E, jnp.int32)
    for j, v in enumerate(vs2):
        cmin2 = jnp.minimum(cmin2, jnp.where(v == m2v, iota + (j * L), E))
    i2v = _shuf_min_i32(cmin2, isc, iota)

    ev = jnp.exp(m2v - m1v)
    w1v = 1.0 / (1.0 + ev)
    w2v = ev * w1v
    return i1v, i2v, w1v, w2v


def _sc_body(cl_hbm, k_hbm, out_hbm,
             lbuf, fsc, isc, idxa0, idxa1, idxb0, idxb1,
             xa0, xa1, xb0, xb1, ob0, ob1,
             sa0, sa1, sb0, sb1, so0, so1):
    wid = lax.axis_index("s") * NC + lax.axis_index("c")
    b = wid // WPB
    part = wid & (WPB - 1)

    pltpu.sync_copy(cl_hbm.at[b], lbuf)
    iota = lax.iota(jnp.int32, L)
    i1v, i2v, w1v, w2v = _routing(lbuf, fsc, isc, iota)

    row_a0 = i1v * ROWS_PER_E + part * (PART // R) + iota
    row_b0 = i2v * ROWS_PER_E + part * (PART // R) + iota
    base_row = part * (PART // D_IN)

    idxa = (idxa0, idxa1)
    idxb = (idxb0, idxb1)
    xa = (xa0, xa1)
    xb = (xb0, xb1)
    ob = (ob0, ob1)
    sa = (sa0, sa1)
    sb = (sb0, sb1)
    so = (so0, so1)

    def issue_gathers(g, s):
        idxa[s][...] = row_a0 + g * NR
        idxb[s][...] = row_b0 + g * NR
        pltpu.async_copy(k_hbm.at[idxa[s].at[pl.ds(0, NR)]], xa[s], sa[s])
        pltpu.async_copy(k_hbm.at[idxb[s].at[pl.ds(0, NR)]], xb[s], sb[s])

    def wait_gathers(s):
        pltpu.make_async_copy(
            k_hbm.at[idxa[s].at[pl.ds(0, NR)]], xa[s], sa[s]).wait()
        pltpu.make_async_copy(
            k_hbm.at[idxb[s].at[pl.ds(0, NR)]], xb[s], sb[s]).wait()

    def wait_owrite(s):
        pltpu.make_async_copy(
            ob[s], out_hbm.at[b, pl.ds(base_row, OROWS), :], so[s]).wait()

    def compute_group(g, s):
        xas, xbs, obs = xa[s], xb[s], ob[s]

        @plsc.parallel_loop(0, GROUP // L, unroll=UNROLL)
        def _(c):
            r = c >> 7
            col = (c & (R // L - 1)) * L
            a0 = xas[r, pl.ds(col, L)]
            a1 = xbs[r, pl.ds(col, L)]
            obs[c >> 6, pl.ds((c & (D_IN // L - 1)) * L, L)] = (
                w1v * a0 + w2v * a1)

        pltpu.async_copy(
            obs, out_hbm.at[b, pl.ds(base_row + g * OROWS, OROWS), :], so[s])

    issue_gathers(0, 0)

    def pair_body(i, _):
        g = i * 2
        wait_gathers(0)
        issue_gathers(g + 1, 1)

        @pl.when(i > 0)
        def _():
            wait_owrite(0)

        compute_group(g, 0)
        wait_gathers(1)

        @pl.when(i < G // 2 - 1)
        def _():
            issue_gathers(g + 2, 0)

        @pl.when(i > 0)
        def _():
            wait_owrite(1)

        compute_group(g + 1, 1)
        return 0

    lax.fori_loop(0, G // 2, pair_body, 0)
    wait_owrite(0)
    wait_owrite(1)


_mesh = plsc.VectorSubcoreMesh(core_axis_name="c", subcore_axis_name="s")

_sc_call = functools.partial(
    pl.kernel,
    mesh=_mesh,
    compiler_params=pltpu.CompilerParams(needs_layout_passes=False),
    out_type=jax.ShapeDtypeStruct((B, D_OUT, D_IN), jnp.float32),
    scratch_types=[
        pltpu.VMEM((E,), jnp.float32),
        pltpu.VMEM((L,), jnp.float32),
        pltpu.VMEM((L,), jnp.int32),
        pltpu.VMEM((L,), jnp.int32),
        pltpu.VMEM((L,), jnp.int32),
        pltpu.VMEM((L,), jnp.int32),
        pltpu.VMEM((L,), jnp.int32),
        pltpu.VMEM((NR, R), jnp.float32),
        pltpu.VMEM((NR, R), jnp.float32),
        pltpu.VMEM((NR, R), jnp.float32),
        pltpu.VMEM((NR, R), jnp.float32),
        pltpu.VMEM((OROWS, D_IN), jnp.float32),
        pltpu.VMEM((OROWS, D_IN), jnp.float32),
        pltpu.SemaphoreType.DMA,
        pltpu.SemaphoreType.DMA,
        pltpu.SemaphoreType.DMA,
        pltpu.SemaphoreType.DMA,
        pltpu.SemaphoreType.DMA,
        pltpu.SemaphoreType.DMA,
    ],
)(_sc_body)


def kernel(config_logits, kernel):
    k_rows = kernel.reshape(E * ROWS_PER_E, R)
    return _sc_call(config_logits, k_rows)


# speedup vs baseline: 8.0124x; 8.0124x over previous
import functools

import jax
import jax.numpy as jnp
from jax import lax
from jax.experimental import pallas as pl
from jax.experimental.pallas import tpu as pltpu
from jax.experimental.pallas import tpu_sc as plsc

E = 64
B = 4
D_OUT = 1024
D_IN = 1024
D = D_OUT * D_IN

L = 16
NC = 2
NS = 16
NW = NC * NS
WPB = NW // B
PART = D // WPB
R = 1024
NR = 16
ROWS_PER_E = D // R
GROUP = NR * R
G = PART // GROUP
OROWS = GROUP // D_IN
UNROLL = 8


def _shuf_max(v, sbuf, iota):
    for sh in (1, 2, 4, 8):
        sbuf[...] = v
        v = jnp.maximum(v, plsc.load_gather(sbuf, [iota ^ sh]))
    return v


def _shuf_min_i32(v, sbuf, iota):
    for sh in (1, 2, 4, 8):
        sbuf[...] = v
        v = jnp.minimum(v, plsc.load_gather(sbuf, [iota ^ sh]))
    return v


def _routing(lbuf, fsc, isc, iota):
    vs = [lbuf[pl.ds(j * L, L)] for j in range(E // L)]

    m = vs[0]
    for v in vs[1:]:
        m = jnp.maximum(m, v)
    m1v = _shuf_max(m, fsc, iota)

    cmin = jnp.full((L,), E, jnp.int32)
    for j, v in enumerate(vs):
        cmin = jnp.minimum(cmin, jnp.where(v == m1v, iota + (j * L), E))
    i1v = _shuf_min_i32(cmin, isc, iota)

    neg_inf = jnp.float32(-jnp.inf)
    vs2 = [jnp.where(iota + (j * L) == i1v, neg_inf, v) for j, v in enumerate(vs)]
    m2 = vs2[0]
    for v in vs2[1:]:
        m2 = jnp.maximum(m2, v)
    m2v = _shuf_max(m2, fsc, iota)

    cmin2 = jnp.full((L,), E, jnp.int32)
    for j, v in enumerate(vs2):
        cmin2 = jnp.minimum(cmin2, jnp.where(v == m2v, iota + (j * L), E))
    i2v = _shuf_min_i32(cmin2, isc, iota)

    ev = jnp.exp(m2v - m1v)
    w1v = 1.0 / (1.0 + ev)
    w2v = ev * w1v
    return i1v, i2v, w1v, w2v


def _sc_body(cl_hbm, k_hbm, out_hbm,
             lbuf, fsc, isc, idxa0, idxa1, idxb0, idxb1,
             xa0, xa1, xb0, xb1, ob0, ob1,
             sa0, sa1, sb0, sb1, so0, so1):
    wid = lax.axis_index("s") * NC + lax.axis_index("c")
    b = wid // WPB
    part = wid & (WPB - 1)

    pltpu.sync_copy(cl_hbm.at[b], lbuf)
    iota = lax.iota(jnp.int32, L)
    i1v, i2v, w1v, w2v = _routing(lbuf, fsc, isc, iota)

    row_a0 = i1v * ROWS_PER_E + part * (PART // R) + iota
    row_b0 = i2v * ROWS_PER_E + part * (PART // R) + iota
    base_row = part * (PART // D_IN)

    idxa = (idxa0, idxa1)
    idxb = (idxb0, idxb1)
    xa = (xa0, xa1)
    xb = (xb0, xb1)
    ob = (ob0, ob1)
    sa = (sa0, sa1)
    sb = (sb0, sb1)
    so = (so0, so1)

    def issue_gathers(g, s):
        idxa[s][...] = row_a0 + g * NR
        idxb[s][...] = row_b0 + g * NR
        pltpu.async_copy(k_hbm.at[idxa[s]], xa[s], sa[s])
        pltpu.async_copy(k_hbm.at[idxb[s]], xb[s], sb[s])

    def wait_gathers(s):
        pltpu.make_async_copy(k_hbm.at[idxa[s]], xa[s], sa[s]).wait()
        pltpu.make_async_copy(k_hbm.at[idxb[s]], xb[s], sb[s]).wait()

    def wait_owrite(s):
        pltpu.make_async_copy(
            ob[s], out_hbm.at[b, pl.ds(base_row, OROWS), :], so[s]).wait()

    def compute_group(g, s):
        xas, xbs, obs = xa[s], xb[s], ob[s]

        @plsc.parallel_loop(0, GROUP // L, unroll=UNROLL)
        def _(c):
            r = c >> 6
            col = (c & (R // L - 1)) * L
            a0 = xas[r, pl.ds(col, L)]
            a1 = xbs[r, pl.ds(col, L)]
            obs[c >> 6, pl.ds((c & (D_IN // L - 1)) * L, L)] = (
                w1v * a0 + w2v * a1)

        pltpu.async_copy(
            obs, out_hbm.at[b, pl.ds(base_row + g * OROWS, OROWS), :], so[s])

    issue_gathers(0, 0)

    def pair_body(i, _):
        g = i * 2
        wait_gathers(0)
        issue_gathers(g + 1, 1)

        @pl.when(i > 0)
        def _():
            wait_owrite(0)

        compute_group(g, 0)
        wait_gathers(1)

        @pl.when(i < G // 2 - 1)
        def _():
            issue_gathers(g + 2, 0)

        @pl.when(i > 0)
        def _():
            wait_owrite(1)

        compute_group(g + 1, 1)
        return 0

    lax.fori_loop(0, G // 2, pair_body, 0)
    wait_owrite(0)
    wait_owrite(1)


_mesh = plsc.VectorSubcoreMesh(core_axis_name="c", subcore_axis_name="s")

_sc_call = functools.partial(
    pl.kernel,
    mesh=_mesh,
    compiler_params=pltpu.CompilerParams(needs_layout_passes=False),
    out_type=jax.ShapeDtypeStruct((B, D_OUT, D_IN), jnp.float32),
    scratch_types=[
        pltpu.VMEM((E,), jnp.float32),
        pltpu.VMEM((L,), jnp.float32),
        pltpu.VMEM((L,), jnp.int32),
        pltpu.VMEM((L,), jnp.int32),
        pltpu.VMEM((L,), jnp.int32),
        pltpu.VMEM((L,), jnp.int32),
        pltpu.VMEM((L,), jnp.int32),
        pltpu.VMEM((NR, R), jnp.float32),
        pltpu.VMEM((NR, R), jnp.float32),
        pltpu.VMEM((NR, R), jnp.float32),
        pltpu.VMEM((NR, R), jnp.float32),
        pltpu.VMEM((OROWS, D_IN), jnp.float32),
        pltpu.VMEM((OROWS, D_IN), jnp.float32),
        pltpu.SemaphoreType.DMA,
        pltpu.SemaphoreType.DMA,
        pltpu.SemaphoreType.DMA,
        pltpu.SemaphoreType.DMA,
        pltpu.SemaphoreType.DMA,
        pltpu.SemaphoreType.DMA,
    ],
)(_sc_body)


def kernel(config_logits, kernel):
    k_rows = kernel.reshape(E * ROWS_PER_E, R)
    return _sc_call(config_logits, k_rows)
